# baseline (speedup 1.0000x reference)
import math

import jax
import jax.numpy as jnp
from jax.experimental import pallas as pl

HEIGHT = 32
WIDTH = 32
INPUT_SIZE = 64
NUM_ITERS = 5
LEARNING_RATE = 0.1
BATCH = 1024
RADIUS = max(HEIGHT / 2.0, WIDTH / 2.0)
TIME_CONSTANT = NUM_ITERS / math.log(RADIUS)
K = HEIGHT * WIDTH
LOG2E = math.log2(math.e)

_BF = jnp.bfloat16
_F32 = jnp.float32


def _split2(a):
    ah = a.astype(_BF)
    al = (a - ah.astype(_F32)).astype(_BF)
    return ah, al


def _split3(a):
    ah = a.astype(_BF)
    r = a - ah.astype(_F32)
    am = r.astype(_BF)
    al = (r - am.astype(_F32)).astype(_BF)
    return ah, am, al


def _mm(lhs, rhs):
    return jax.lax.dot_general(lhs, rhs, (((1,), (0,)), ((), ())),
                               preferred_element_type=_F32)


def _som_body(x1h_ref, x1l_ref, xth_ref, xtm_ref, xtl_ref, w_ref, out_ref):
    x1h, x1l = x1h_ref[:], x1l_ref[:]
    xth, xtm, xtl = xth_ref[:], xtm_ref[:], xtl_ref[:]
    w = w_ref[:]

    krow = jax.lax.broadcasted_iota(jnp.int32, (K, 1), 0)
    ki = (krow >> 5).astype(_F32)
    kj = (krow & 31).astype(_F32)

    bmu = None
    for i in range(NUM_ITERS):
        lr = LEARNING_RATE * math.exp(-i / NUM_ITERS)
        nr = RADIUS * math.exp(-i / TIME_CONSTANT)
        nr2 = nr * nr

        wn = jnp.sum(w * w, axis=1, keepdims=True)
        wh, wm, wl = _split3(w)
        dots = (_mm(wh, xth) + _mm(wh, xtm) + _mm(wm, xth)
                + _mm(wh, xtl) + _mm(wl, xth) + _mm(wm, xtm))
        score = wn + dots

        cmin = jnp.min(score, axis=0, keepdims=True)
        bmu = jnp.min(jnp.where(score == cmin, krow, K), axis=0,
                      keepdims=True).astype(jnp.int32)

        c = -0.5 * LOG2E / nr2
        bias = math.log2(lr)
        bif = (bmu >> 5).astype(_F32)
        bjf = (bmu & 31).astype(_F32)
        bi2 = bif * (-2.0 * c)
        bj2 = bjf * (-2.0 * c)
        q = (bif * bif + bjf * bjf) * c
        r = (ki * ki + kj * kj) * c + bias
        t = (ki * bi2 + r) + kj * bj2 + q
        d2max = math.ceil(nr2) - 1 if float(nr2).is_integer() else math.floor(nr2)
        thresh = c * (d2max + 0.5) + bias
        eff = jnp.where(t > thresh, jnp.exp2(t), 0.0)

        eh, el = _split2(eff)
        us = _mm(eh, x1h) + _mm(eh, x1l) + _mm(el, x1h)
        u = us[:, :INPUT_SIZE]
        s = us[:, INPUT_SIZE:INPUT_SIZE + 1]
        w = w * (1.0 - s * (1.0 / BATCH)) + u * (1.0 / BATCH)

    bmu_col = jnp.transpose(bmu, (1, 0))
    kcols = jax.lax.broadcasted_iota(jnp.int32, (1, K), 1)
    onehot = (kcols == bmu_col).astype(_BF)
    wh, wm, wl = _split3(w)
    out_ref[:] = _mm(onehot, wh) + _mm(onehot, wm) + _mm(onehot, wl)


def kernel(inputs, weights, locations):
    del locations
    ones_col = jnp.ones((BATCH, 1), _F32)
    x1 = jnp.concatenate([inputs, ones_col], axis=1)
    x1h = x1.astype(_BF)
    x1l = (x1 - x1h.astype(_F32)).astype(_BF)
    xt2 = jnp.transpose(-2.0 * inputs, (1, 0))
    xth = xt2.astype(_BF)
    r = xt2 - xth.astype(_F32)
    xtm = r.astype(_BF)
    xtl = (r - xtm.astype(_F32)).astype(_BF)
    return pl.pallas_call(
        _som_body,
        out_shape=jax.ShapeDtypeStruct((BATCH, INPUT_SIZE), _F32),
    )(x1h, x1l, xth, xtm, xtl, weights)


# speedup vs baseline: 1.1567x; 1.1567x over previous
import math

import jax
import jax.numpy as jnp
from jax.experimental import pallas as pl

HEIGHT = 32
WIDTH = 32
INPUT_SIZE = 64
NUM_ITERS = 5
LEARNING_RATE = 0.1
BATCH = 1024
RADIUS = max(HEIGHT / 2.0, WIDTH / 2.0)
TIME_CONSTANT = NUM_ITERS / math.log(RADIUS)
K = HEIGHT * WIDTH
LOG2E = math.log2(math.e)

_BF = jnp.bfloat16
_F32 = jnp.float32


def _split2(a):
    ah = a.astype(_BF)
    al = (a - ah.astype(_F32)).astype(_BF)
    return ah, al


def _split3(a):
    ah = a.astype(_BF)
    r = a - ah.astype(_F32)
    am = r.astype(_BF)
    al = (r - am.astype(_F32)).astype(_BF)
    return ah, am, al


def _mm(lhs, rhs):
    return jax.lax.dot_general(lhs, rhs, (((1,), (0,)), ((), ())),
                               preferred_element_type=_F32)


def _som_body(x1h_ref, x1l_ref, xt2_ref, w_ref, out_ref):
    x1h, x1l = x1h_ref[:], x1l_ref[:]
    xt2 = xt2_ref[:]
    w = w_ref[:]

    krow = jax.lax.broadcasted_iota(jnp.int32, (K, 1), 0)
    ki = (krow >> 5).astype(_F32)
    kj = (krow & 31).astype(_F32)

    bmu = None
    for i in range(NUM_ITERS):
        lr = LEARNING_RATE * math.exp(-i / NUM_ITERS)
        nr = RADIUS * math.exp(-i / TIME_CONSTANT)
        nr2 = nr * nr

        wn = jnp.sum(w * w, axis=1, keepdims=True)
        dots = jax.lax.dot_general(w, xt2, (((1,), (0,)), ((), ())),
                                   preferred_element_type=_F32,
                                   precision=jax.lax.Precision.HIGHEST)
        score = wn + dots

        cmin = jnp.min(score, axis=0, keepdims=True)
        bmu = jnp.min(jnp.where(score == cmin, krow, K), axis=0,
                      keepdims=True).astype(jnp.int32)

        c = -0.5 * LOG2E / nr2
        bias = math.log2(lr)
        bif = (bmu >> 5).astype(_F32)
        bjf = (bmu & 31).astype(_F32)
        bi2 = bif * (-2.0 * c)
        bj2 = bjf * (-2.0 * c)
        q = (bif * bif + bjf * bjf) * c
        r = (ki * ki + kj * kj) * c + bias
        t = (ki * bi2 + r) + kj * bj2 + q
        d2max = math.ceil(nr2) - 1 if float(nr2).is_integer() else math.floor(nr2)
        thresh = c * (d2max + 0.5) + bias
        eff = jnp.where(t > thresh, jnp.exp2(t), 0.0)

        eh, el = _split2(eff)
        us = _mm(eh, x1h) + _mm(eh, x1l) + _mm(el, x1h)
        u = us[:, :INPUT_SIZE]
        s = us[:, INPUT_SIZE:INPUT_SIZE + 1]
        w = w * (1.0 - s * (1.0 / BATCH)) + u * (1.0 / BATCH)

    bmu_col = jnp.transpose(bmu, (1, 0))
    kcols = jax.lax.broadcasted_iota(jnp.int32, (1, K), 1)
    onehot = (kcols == bmu_col).astype(_BF)
    wh, wm, wl = _split3(w)
    out_ref[:] = _mm(onehot, wh) + _mm(onehot, wm) + _mm(onehot, wl)


def kernel(inputs, weights, locations):
    del locations
    ones_col = jnp.ones((BATCH, 1), _F32)
    x1 = jnp.concatenate([inputs, ones_col], axis=1)
    x1h = x1.astype(_BF)
    x1l = (x1 - x1h.astype(_F32)).astype(_BF)
    xt2 = jnp.transpose(-2.0 * inputs, (1, 0))
    return pl.pallas_call(
        _som_body,
        out_shape=jax.ShapeDtypeStruct((BATCH, INPUT_SIZE), _F32),
    )(x1h, x1l, xt2, weights)


# speedup vs baseline: 1.1580x; 1.0011x over previous
import math

import jax
import jax.numpy as jnp
from jax.experimental import pallas as pl

HEIGHT = 32
WIDTH = 32
INPUT_SIZE = 64
NUM_ITERS = 5
LEARNING_RATE = 0.1
BATCH = 1024
RADIUS = max(HEIGHT / 2.0, WIDTH / 2.0)
TIME_CONSTANT = NUM_ITERS / math.log(RADIUS)
K = HEIGHT * WIDTH
LOG2E = math.log2(math.e)

_BF = jnp.bfloat16
_F32 = jnp.float32


def _split2(a):
    ah = a.astype(_BF)
    al = (a - ah.astype(_F32)).astype(_BF)
    return ah, al


def _split3(a):
    ah = a.astype(_BF)
    r = a - ah.astype(_F32)
    am = r.astype(_BF)
    al = (r - am.astype(_F32)).astype(_BF)
    return ah, am, al


def _mm(lhs, rhs):
    return jax.lax.dot_general(lhs, rhs, (((1,), (0,)), ((), ())),
                               preferred_element_type=_F32)


def _som_body(x1h_ref, x1l_ref, xt2_ref, w_ref, out_ref):
    x1h, x1l = x1h_ref[:], x1l_ref[:]
    xt2 = xt2_ref[:]
    w = w_ref[:]

    krow = jax.lax.broadcasted_iota(jnp.int32, (K, 1), 0)
    ki = (krow >> 5).astype(_F32)
    kj = (krow & 31).astype(_F32)

    bmu = None
    for i in range(NUM_ITERS):
        lr = LEARNING_RATE * math.exp(-i / NUM_ITERS)
        nr = RADIUS * math.exp(-i / TIME_CONSTANT)
        nr2 = nr * nr

        wn = jnp.sum(w * w, axis=1, keepdims=True)
        dots = jax.lax.dot_general(w, xt2, (((1,), (0,)), ((), ())),
                                   preferred_element_type=_F32,
                                   precision=jax.lax.Precision.HIGHEST)
        score = wn + dots

        cmin = jnp.min(score, axis=0, keepdims=True)
        bmu = jnp.min(jnp.where(score == cmin, krow, K), axis=0,
                      keepdims=True).astype(jnp.int32)

        c = -0.5 * LOG2E / nr2
        bias = math.log2(lr)
        bif = (bmu >> 5).astype(_F32)
        bjf = (bmu & 31).astype(_F32)
        bi2 = bif * (-2.0 * c)
        bj2 = bjf * (-2.0 * c)
        q = (bif * bif + bjf * bjf) * c + bias
        r = (ki * ki + kj * kj) * c
        t = (ki * bi2 + r) + (kj * bj2 + q)
        d2max = math.ceil(nr2) - 1 if float(nr2).is_integer() else math.floor(nr2)
        thresh = c * (d2max + 0.5) + bias
        eff = jnp.where(t > thresh, jnp.exp2(t), 0.0)

        eh, el = _split2(eff)
        us = _mm(eh, x1h) + _mm(eh, x1l) + _mm(el, x1h)
        u = us[:, :INPUT_SIZE]
        s = us[:, INPUT_SIZE:INPUT_SIZE + 1]
        w = w * (1.0 - s * (1.0 / BATCH)) + u * (1.0 / BATCH)

    bmu_col = jnp.transpose(bmu, (1, 0))
    kcols = jax.lax.broadcasted_iota(jnp.int32, (1, K), 1)
    onehot = (kcols == bmu_col).astype(_BF)
    wh, wm, wl = _split3(w)
    out_ref[:] = _mm(onehot, wh) + _mm(onehot, wm) + _mm(onehot, wl)


def kernel(inputs, weights, locations):
    del locations
    ones_col = jnp.ones((BATCH, 1), _F32)
    x1 = jnp.concatenate([inputs, ones_col], axis=1)
    x1h = x1.astype(_BF)
    x1l = (x1 - x1h.astype(_F32)).astype(_BF)
    xt2 = jnp.transpose(-2.0 * inputs, (1, 0))
    return pl.pallas_call(
        _som_body,
        out_shape=jax.ShapeDtypeStruct((BATCH, INPUT_SIZE), _F32),
    )(x1h, x1l, xt2, weights)


# speedup vs baseline: 1.1940x; 1.0310x over previous
import math

import jax
import jax.numpy as jnp
from jax.experimental import pallas as pl

HEIGHT = 32
WIDTH = 32
INPUT_SIZE = 64
NUM_ITERS = 5
LEARNING_RATE = 0.1
BATCH = 1024
RADIUS = max(HEIGHT / 2.0, WIDTH / 2.0)
TIME_CONSTANT = NUM_ITERS / math.log(RADIUS)
K = HEIGHT * WIDTH
LOG2E = math.log2(math.e)

_BF = jnp.bfloat16
_F32 = jnp.float32


def _split2(a):
    ah = a.astype(_BF)
    al = (a - ah.astype(_F32)).astype(_BF)
    return ah, al


def _split3(a):
    ah = a.astype(_BF)
    r = a - ah.astype(_F32)
    am = r.astype(_BF)
    al = (r - am.astype(_F32)).astype(_BF)
    return ah, am, al


def _mm(lhs, rhs):
    return jax.lax.dot_general(lhs, rhs, (((1,), (0,)), ((), ())),
                               preferred_element_type=_F32)


def _som_body(x1h_ref, x1l_ref, xt2_ref, w_ref, out_ref):
    x1h, x1l = x1h_ref[:], x1l_ref[:]
    xt2 = xt2_ref[:]
    w = w_ref[:]

    krow = jax.lax.broadcasted_iota(jnp.int32, (K, 1), 0)
    ki = (krow >> 5).astype(_F32)
    kj = (krow & 31).astype(_F32)

    bmu = None
    for i in range(NUM_ITERS):
        lr = LEARNING_RATE * math.exp(-i / NUM_ITERS)
        nr = RADIUS * math.exp(-i / TIME_CONSTANT)
        nr2 = nr * nr

        wn = jnp.sum(w * w, axis=1, keepdims=True)
        dots = jax.lax.dot_general(w, xt2, (((1,), (0,)), ((), ())),
                                   preferred_element_type=_F32,
                                   precision=jax.lax.Precision.HIGHEST)
        score = wn + dots

        bmu = jnp.argmin(score, axis=0).astype(jnp.int32)[None, :]

        c = -0.5 * LOG2E / nr2
        bias = math.log2(lr)
        bif = (bmu >> 5).astype(_F32)
        bjf = (bmu & 31).astype(_F32)
        bi2 = bif * (-2.0 * c)
        bj2 = bjf * (-2.0 * c)
        q = (bif * bif + bjf * bjf) * c + bias
        r = (ki * ki + kj * kj) * c
        t = (ki * bi2 + r) + (kj * bj2 + q)
        d2max = math.ceil(nr2) - 1 if float(nr2).is_integer() else math.floor(nr2)
        thresh = c * (d2max + 0.5) + bias
        eff = jnp.where(t > thresh, jnp.exp2(t), 0.0)

        eh, el = _split2(eff)
        us = _mm(eh, x1h) + _mm(eh, x1l) + _mm(el, x1h)
        u = us[:, :INPUT_SIZE]
        s = us[:, INPUT_SIZE:INPUT_SIZE + 1]
        w = w * (1.0 - s * (1.0 / BATCH)) + u * (1.0 / BATCH)

    bmu_col = jnp.transpose(bmu, (1, 0))
    kcols = jax.lax.broadcasted_iota(jnp.int32, (1, K), 1)
    onehot = (kcols == bmu_col).astype(_BF)
    wh, wm, wl = _split3(w)
    out_ref[:] = _mm(onehot, wh) + _mm(onehot, wm) + _mm(onehot, wl)


def kernel(inputs, weights, locations):
    del locations
    ones_col = jnp.ones((BATCH, 1), _F32)
    x1 = jnp.concatenate([inputs, ones_col], axis=1)
    x1h = x1.astype(_BF)
    x1l = (x1 - x1h.astype(_F32)).astype(_BF)
    xt2 = jnp.transpose(-2.0 * inputs, (1, 0))
    return pl.pallas_call(
        _som_body,
        out_shape=jax.ShapeDtypeStruct((BATCH, INPUT_SIZE), _F32),
    )(x1h, x1l, xt2, weights)
